# Initial kernel scaffold; baseline (speedup 1.0000x reference)
import jax
import jax.numpy as jnp
from jax.experimental import pallas as pl


def kernel(feat, edge_index, W0, b0, W1, b1, alpha):
    raise NotImplementedError("write your pallas kernel here")


# speedup vs baseline: 5.0226x; 5.0226x over previous
import functools

import jax
import jax.numpy as jnp
from jax import lax
from jax.experimental import pallas as pl
from jax.experimental.pallas import tpu as pltpu
from jax.experimental.pallas import tpu_sc as plsc

N = 10000
E = 320000
D = 128

NC = 2
NS = 16
NW = NC * NS
N_PAD = 10240
EW = E // NW
CH = 80
NCHUNK = EW // CH
SL = N_PAD // NW
RPT = N // NS
BN = 1000

_LANES = 16


def _sc_mesh():
    return plsc.VectorSubcoreMesh(core_axis_name="c", subcore_axis_name="s")


def _degree_body(src_hbm, dst_hbm, parts_hbm, idx_v, hist_v):
    c = lax.axis_index("c")
    s = lax.axis_index("s")
    wid = c * NS + s
    base = wid * EW
    ones = jnp.ones((_LANES,), jnp.float32)
    zeros = jnp.zeros((_LANES,), jnp.float32)
    for a, edges in ((0, src_hbm), (1, dst_hbm)):
        def zero_body(i, carry):
            hist_v[pl.ds(i * _LANES, _LANES)] = zeros
            return carry
        lax.fori_loop(0, N_PAD // _LANES, zero_body, 0)
        pltpu.sync_copy(edges.at[pl.ds(base, EW)], idx_v)

        def acc_body(t, carry):
            idx = idx_v[pl.ds(t * _LANES, _LANES)]
            plsc.addupdate_scatter(hist_v, [idx], ones)
            return carry
        lax.fori_loop(0, EW // _LANES, acc_body, 0)
        pltpu.sync_copy(hist_v, parts_hbm.at[a * NW + wid])


@jax.jit
def _degree_call(src, dst):
    f = functools.partial(
        pl.kernel,
        out_type=jax.ShapeDtypeStruct((2 * NW, N_PAD), jnp.float32),
        mesh=_sc_mesh(),
        scratch_types=[
            pltpu.VMEM((EW,), jnp.int32),
            pltpu.VMEM((N_PAD,), jnp.float32),
        ],
        compiler_params=pltpu.CompilerParams(
            needs_layout_passes=False, use_tc_tiling_on_sc=False),
    )(_degree_body)
    return f(src, dst)


def _norm_body(parts_hbm, deg2d_hbm, pbuf, acc, rowbuf, sem):
    c = lax.axis_index("c")
    s = lax.axis_index("s")
    wid = c * NS + s
    n0 = wid * SL
    zeros = jnp.zeros((_LANES,), jnp.float32)

    def load_body(t, carry):
        pltpu.async_copy(parts_hbm.at[t, pl.ds(n0, SL)], pbuf.at[t], sem)
        return carry
    lax.fori_loop(0, 2 * NW, load_body, 0)

    def wait_body(t, carry):
        pltpu.make_async_copy(parts_hbm.at[t, pl.ds(n0, SL)], pbuf.at[t], sem).wait()
        return carry
    lax.fori_loop(0, 2 * NW, wait_body, 0)

    for a in (0, 1):
        def zero_body(k, carry):
            acc[pl.ds(k * _LANES, _LANES)] = zeros
            return carry
        lax.fori_loop(0, SL // _LANES, zero_body, 0)

        def red_body(i, carry):
            t = i // (SL // _LANES)
            k = i % (SL // _LANES)
            acc[pl.ds(k * _LANES, _LANES)] = (
                acc[pl.ds(k * _LANES, _LANES)]
                + pbuf[a * NW + t, pl.ds(k * _LANES, _LANES)]
            )
            return carry
        lax.fori_loop(0, NW * (SL // _LANES), red_body, 0)

        def bcast_body(g, carry):
            vec16 = acc[pl.ds(g * _LANES, _LANES)]
            for j in range(_LANES):
                vec = lax.broadcast(vec16[j], (_LANES,))
                for k in range(D // _LANES):
                    rowbuf[g * _LANES + j, pl.ds(k * _LANES, _LANES)] = vec
            return carry
        lax.fori_loop(0, SL // _LANES, bcast_body, 0)
        pltpu.sync_copy(rowbuf, deg2d_hbm.at[a, pl.ds(n0, SL)])


@jax.jit
def _norm_call(parts):
    f = functools.partial(
        pl.kernel,
        out_type=jax.ShapeDtypeStruct((2, N_PAD, D), jnp.float32),
        mesh=_sc_mesh(),
        scratch_types=[
            pltpu.VMEM((2 * NW, SL), jnp.float32),
            pltpu.VMEM((SL,), jnp.float32),
            pltpu.VMEM((SL, D), jnp.float32),
            pltpu.SemaphoreType.DMA,
        ],
        compiler_params=pltpu.CompilerParams(use_tc_tiling_on_sc=False),
    )(_norm_body)
    return f(parts)


def _agg_body(x_hbm, src_hbm, dst_hbm, zrows_hbm, parts_hbm,
              sidx, didx, rows, acc_sh, gsem):
    c = lax.axis_index("c")
    s = lax.axis_index("s")
    wid = c * NS + s
    base = wid * EW

    pltpu.sync_copy(zrows_hbm, acc_sh.at[pl.ds(s * RPT, RPT)])
    plsc.subcore_barrier()

    def chunk_body(j, carry):
        off = base + j * CH
        pltpu.sync_copy(src_hbm.at[pl.ds(off, CH)], sidx)
        pltpu.sync_copy(dst_hbm.at[pl.ds(off, CH)], didx)
        pltpu.async_copy(x_hbm.at[sidx], rows, gsem).wait()
        pltpu.sync_copy(rows, acc_sh.at[didx], add=True)
        return carry
    lax.fori_loop(0, NCHUNK, chunk_body, 0)

    plsc.subcore_barrier()
    pltpu.sync_copy(acc_sh.at[pl.ds(s * RPT, RPT)],
                    parts_hbm.at[c, pl.ds(s * RPT, RPT)])


@jax.jit
def _agg_call(x, src, dst, zrows):
    f = functools.partial(
        pl.kernel,
        out_type=jax.ShapeDtypeStruct((NC, N, D), jnp.float32),
        mesh=_sc_mesh(),
        scratch_types=[
            pltpu.VMEM((CH,), jnp.int32),
            pltpu.VMEM((CH,), jnp.int32),
            pltpu.VMEM((CH, D), jnp.float32),
            pltpu.VMEM_SHARED((N, D), jnp.float32),
            pltpu.SemaphoreType.DMA,
        ],
        compiler_params=pltpu.CompilerParams(use_tc_tiling_on_sc=False),
    )(_agg_body)
    return f(x, src, dst, zrows)


def _l0_body(feat_ref, w0_ref, dout_ref, x0_ref):
    ns = lax.rsqrt(jnp.maximum(dout_ref[0], 1.0))
    y = jnp.dot(feat_ref[...], w0_ref[...],
                preferred_element_type=jnp.float32,
                precision=lax.Precision.HIGHEST)
    x0_ref[...] = y * ns


@jax.jit
def _l0_call(feat, W0, deg2d):
    return pl.pallas_call(
        _l0_body,
        grid=(N // BN,),
        in_specs=[
            pl.BlockSpec((BN, D), lambda i: (i, 0)),
            pl.BlockSpec((D, D), lambda i: (0, 0)),
            pl.BlockSpec((1, BN, D), lambda i: (0, i, 0)),
        ],
        out_specs=pl.BlockSpec((BN, D), lambda i: (i, 0)),
        out_shape=jax.ShapeDtypeStruct((N, D), jnp.float32),
    )(feat, W0, deg2d)


def _mid_body(p0_ref, p1_ref, din_ref, dout_ref, b0_ref, alpha_ref, w1_ref,
              x1_ref, hg_ref):
    @pl.when(pl.program_id(0) == 0)
    def _():
        hg_ref[...] = jnp.zeros_like(hg_ref)

    nd = lax.rsqrt(jnp.maximum(din_ref[0], 1.0))
    z = (p0_ref[0] + p1_ref[0]) * nd + b0_ref[...]
    a = alpha_ref[0, 0]
    h = jnp.where(z >= 0, z, a * z)
    hg_ref[...] += jnp.sum(h, axis=0, keepdims=True)
    ns = lax.rsqrt(jnp.maximum(dout_ref[0], 1.0))
    x1_ref[...] = jnp.dot(h, w1_ref[...],
                          preferred_element_type=jnp.float32,
                          precision=lax.Precision.HIGHEST) * ns


@jax.jit
def _mid_call(p, deg2d, b0, alpha, W1):
    return pl.pallas_call(
        _mid_body,
        grid=(N // BN,),
        in_specs=[
            pl.BlockSpec((1, BN, D), lambda i: (0, i, 0)),
            pl.BlockSpec((1, BN, D), lambda i: (1, i, 0)),
            pl.BlockSpec((1, BN, D), lambda i: (1, i, 0)),
            pl.BlockSpec((1, BN, D), lambda i: (0, i, 0)),
            pl.BlockSpec((1, D), lambda i: (0, 0)),
            pl.BlockSpec((1, 1), lambda i: (0, 0)),
            pl.BlockSpec((D, D), lambda i: (0, 0)),
        ],
        out_specs=[
            pl.BlockSpec((BN, D), lambda i: (i, 0)),
            pl.BlockSpec((1, D), lambda i: (0, 0)),
        ],
        out_shape=[
            jax.ShapeDtypeStruct((N, D), jnp.float32),
            jax.ShapeDtypeStruct((1, D), jnp.float32),
        ],
    )(p, p, deg2d, deg2d, b0, alpha, W1)


def _out_body(q0_ref, q1_ref, din_ref, b1_ref, alpha_ref, h2_ref, hg_ref):
    @pl.when(pl.program_id(0) == 0)
    def _():
        hg_ref[...] = jnp.zeros_like(hg_ref)

    nd = lax.rsqrt(jnp.maximum(din_ref[0], 1.0))
    z = (q0_ref[0] + q1_ref[0]) * nd + b1_ref[...]
    a = alpha_ref[0, 0]
    h = jnp.where(z >= 0, z, a * z)
    h2_ref[...] = h
    hg_ref[...] += jnp.sum(h, axis=0, keepdims=True)


@jax.jit
def _out_call(q, deg2d, b1, alpha):
    return pl.pallas_call(
        _out_body,
        grid=(N // BN,),
        in_specs=[
            pl.BlockSpec((1, BN, D), lambda i: (0, i, 0)),
            pl.BlockSpec((1, BN, D), lambda i: (1, i, 0)),
            pl.BlockSpec((1, BN, D), lambda i: (1, i, 0)),
            pl.BlockSpec((1, D), lambda i: (0, 0)),
            pl.BlockSpec((1, 1), lambda i: (0, 0)),
        ],
        out_specs=[
            pl.BlockSpec((BN, D), lambda i: (i, 0)),
            pl.BlockSpec((1, D), lambda i: (0, 0)),
        ],
        out_shape=[
            jax.ShapeDtypeStruct((N, D), jnp.float32),
            jax.ShapeDtypeStruct((1, D), jnp.float32),
        ],
    )(q, q, deg2d, b1, alpha)


def kernel(feat, edge_index, W0, b0, W1, b1, alpha):
    src = edge_index[0]
    dst = edge_index[1]
    parts = _degree_call(src, dst)
    deg2d = _norm_call(parts)
    x0 = _l0_call(feat, W0, deg2d)
    zrows = jnp.zeros((RPT, D), jnp.float32)
    p = _agg_call(x0, src, dst, zrows)
    x1, hg0 = _mid_call(p, deg2d, b0.reshape(1, D), alpha.reshape(1, 1), W1)
    q = _agg_call(x1, src, dst, zrows)
    h2, hg1 = _out_call(q, deg2d, b1.reshape(1, D), alpha.reshape(1, 1))
    hg = jnp.concatenate((hg0, hg1), axis=-1)
    return (h2, hg)


# speedup vs baseline: 10.4181x; 2.0742x over previous
import functools

import jax
import jax.numpy as jnp
from jax import lax
from jax.experimental import pallas as pl
from jax.experimental.pallas import tpu as pltpu
from jax.experimental.pallas import tpu_sc as plsc

N = 10000
E = 320000
D = 128

NC = 2
NS = 16
NW = NC * NS
N_PAD = 10240
EW = E // NW
CH = 80
NCHUNK = EW // CH
SL = N_PAD // NW
RPT = N // NS
BN = 1000

_LANES = 16


def _sc_mesh():
    return plsc.VectorSubcoreMesh(core_axis_name="c", subcore_axis_name="s")


def _degree_body(src_hbm, dst_hbm, parts_hbm, idx_v, hist_v):
    c = lax.axis_index("c")
    s = lax.axis_index("s")
    wid = c * NS + s
    base = wid * EW
    ones = jnp.ones((_LANES,), jnp.float32)
    zeros = jnp.zeros((_LANES,), jnp.float32)
    for a, edges in ((0, src_hbm), (1, dst_hbm)):
        def zero_body(i, carry):
            hist_v[pl.ds(i * _LANES, _LANES)] = zeros
            return carry
        lax.fori_loop(0, N_PAD // _LANES, zero_body, 0)
        pltpu.sync_copy(edges.at[pl.ds(base, EW)], idx_v)

        def acc_body(t, carry):
            idx = idx_v[pl.ds(t * _LANES, _LANES)]
            plsc.addupdate_scatter(hist_v, [idx], ones)
            return carry
        lax.fori_loop(0, EW // _LANES, acc_body, 0)
        pltpu.sync_copy(hist_v, parts_hbm.at[a * NW + wid])


@jax.jit
def _degree_call(src, dst):
    f = functools.partial(
        pl.kernel,
        out_type=jax.ShapeDtypeStruct((2 * NW, N_PAD), jnp.float32),
        mesh=_sc_mesh(),
        scratch_types=[
            pltpu.VMEM((EW,), jnp.int32),
            pltpu.VMEM((N_PAD,), jnp.float32),
        ],
        compiler_params=pltpu.CompilerParams(
            needs_layout_passes=False, use_tc_tiling_on_sc=False),
    )(_degree_body)
    return f(src, dst)


def _norm_body(parts_hbm, deg2d_hbm, pbuf, acc, rowbuf, sem):
    c = lax.axis_index("c")
    s = lax.axis_index("s")
    wid = c * NS + s
    n0 = wid * SL
    zeros = jnp.zeros((_LANES,), jnp.float32)

    def load_body(t, carry):
        pltpu.async_copy(parts_hbm.at[t, pl.ds(n0, SL)], pbuf.at[t], sem)
        return carry
    lax.fori_loop(0, 2 * NW, load_body, 0)

    def wait_body(t, carry):
        pltpu.make_async_copy(parts_hbm.at[t, pl.ds(n0, SL)], pbuf.at[t], sem).wait()
        return carry
    lax.fori_loop(0, 2 * NW, wait_body, 0)

    for a in (0, 1):
        def zero_body(k, carry):
            acc[pl.ds(k * _LANES, _LANES)] = zeros
            return carry
        lax.fori_loop(0, SL // _LANES, zero_body, 0)

        def red_body(i, carry):
            t = i // (SL // _LANES)
            k = i % (SL // _LANES)
            acc[pl.ds(k * _LANES, _LANES)] = (
                acc[pl.ds(k * _LANES, _LANES)]
                + pbuf[a * NW + t, pl.ds(k * _LANES, _LANES)]
            )
            return carry
        lax.fori_loop(0, NW * (SL // _LANES), red_body, 0)

        def bcast_body(g, carry):
            vec16 = acc[pl.ds(g * _LANES, _LANES)]
            for j in range(_LANES):
                vec = lax.broadcast(vec16[j], (_LANES,))
                for k in range(D // _LANES):
                    rowbuf[g * _LANES + j, pl.ds(k * _LANES, _LANES)] = vec
            return carry
        lax.fori_loop(0, SL // _LANES, bcast_body, 0)
        pltpu.sync_copy(rowbuf, deg2d_hbm.at[a, pl.ds(n0, SL)])


@jax.jit
def _norm_call(parts):
    f = functools.partial(
        pl.kernel,
        out_type=jax.ShapeDtypeStruct((2, N_PAD, D), jnp.float32),
        mesh=_sc_mesh(),
        scratch_types=[
            pltpu.VMEM((2 * NW, SL), jnp.float32),
            pltpu.VMEM((SL,), jnp.float32),
            pltpu.VMEM((SL, D), jnp.float32),
            pltpu.SemaphoreType.DMA,
        ],
        compiler_params=pltpu.CompilerParams(use_tc_tiling_on_sc=False),
    )(_norm_body)
    return f(parts)


def _agg_body(x_hbm, src2d_hbm, dst2d_hbm, zrows_hbm, parts_hbm,
              sidx2d, didx2d, rows0, rows1, acc_sh, gsem0, gsem1):
    c = lax.axis_index("c")
    s = lax.axis_index("s")
    wid = c * NS + s

    pltpu.sync_copy(zrows_hbm, acc_sh.at[pl.ds(s * RPT, RPT)])

    pltpu.sync_copy(src2d_hbm.at[pl.ds(wid * NCHUNK, NCHUNK)], sidx2d)
    pltpu.sync_copy(dst2d_hbm.at[pl.ds(wid * NCHUNK, NCHUNK)], didx2d)
    plsc.subcore_barrier()

    def start(j, rbuf, sem):
        pltpu.async_copy(x_hbm.at[sidx2d.at[j]], rbuf, sem)

    def wait(rbuf, sem):
        pltpu.make_async_copy(x_hbm.at[sidx2d.at[0]], rbuf, sem).wait()

    def scatter(j, rbuf):
        pltpu.sync_copy(rbuf, acc_sh.at[didx2d.at[j]], add=True)

    start(0, rows0, gsem0)
    start(1, rows1, gsem1)

    def body(i, carry):
        c0 = 2 * i
        wait(rows0, gsem0)
        scatter(c0, rows0)
        start(c0 + 2, rows0, gsem0)

        c1 = 2 * i + 1
        wait(rows1, gsem1)
        scatter(c1, rows1)

        @pl.when(c1 + 2 < NCHUNK)
        def _():
            start(c1 + 2, rows1, gsem1)
        return carry
    lax.fori_loop(0, (NCHUNK - 1) // 2, body, 0)

    wait(rows0, gsem0)
    scatter(NCHUNK - 1, rows0)

    plsc.subcore_barrier()
    pltpu.sync_copy(acc_sh.at[pl.ds(s * RPT, RPT)],
                    parts_hbm.at[c, pl.ds(s * RPT, RPT)])


@jax.jit
def _agg_call(x, src, dst, zrows):
    f = functools.partial(
        pl.kernel,
        out_type=jax.ShapeDtypeStruct((NC, N, D), jnp.float32),
        mesh=_sc_mesh(),
        scratch_types=[
            pltpu.VMEM((NCHUNK, CH), jnp.int32),
            pltpu.VMEM((NCHUNK, CH), jnp.int32),
            pltpu.VMEM((CH, D), jnp.float32),
            pltpu.VMEM((CH, D), jnp.float32),
            pltpu.VMEM_SHARED((N, D), jnp.float32),
            pltpu.SemaphoreType.DMA,
            pltpu.SemaphoreType.DMA,
        ],
        compiler_params=pltpu.CompilerParams(use_tc_tiling_on_sc=False),
    )(_agg_body)
    return f(x, src.reshape(E // CH, CH), dst.reshape(E // CH, CH), zrows)


def _l0_body(feat_ref, w0_ref, dout_ref, x0_ref):
    ns = lax.rsqrt(jnp.maximum(dout_ref[0], 1.0))
    y = jnp.dot(feat_ref[...], w0_ref[...],
                preferred_element_type=jnp.float32,
                precision=lax.Precision.HIGHEST)
    x0_ref[...] = y * ns


@jax.jit
def _l0_call(feat, W0, deg2d):
    return pl.pallas_call(
        _l0_body,
        grid=(N // BN,),
        in_specs=[
            pl.BlockSpec((BN, D), lambda i: (i, 0)),
            pl.BlockSpec((D, D), lambda i: (0, 0)),
            pl.BlockSpec((1, BN, D), lambda i: (0, i, 0)),
        ],
        out_specs=pl.BlockSpec((BN, D), lambda i: (i, 0)),
        out_shape=jax.ShapeDtypeStruct((N, D), jnp.float32),
    )(feat, W0, deg2d)


def _mid_body(p0_ref, p1_ref, din_ref, dout_ref, b0_ref, alpha_ref, w1_ref,
              x1_ref, hg_ref):
    @pl.when(pl.program_id(0) == 0)
    def _():
        hg_ref[...] = jnp.zeros_like(hg_ref)

    nd = lax.rsqrt(jnp.maximum(din_ref[0], 1.0))
    z = (p0_ref[0] + p1_ref[0]) * nd + b0_ref[...]
    a = alpha_ref[0, 0]
    h = jnp.where(z >= 0, z, a * z)
    hg_ref[...] += jnp.sum(h, axis=0, keepdims=True)
    ns = lax.rsqrt(jnp.maximum(dout_ref[0], 1.0))
    x1_ref[...] = jnp.dot(h, w1_ref[...],
                          preferred_element_type=jnp.float32,
                          precision=lax.Precision.HIGHEST) * ns


@jax.jit
def _mid_call(p, deg2d, b0, alpha, W1):
    return pl.pallas_call(
        _mid_body,
        grid=(N // BN,),
        in_specs=[
            pl.BlockSpec((1, BN, D), lambda i: (0, i, 0)),
            pl.BlockSpec((1, BN, D), lambda i: (1, i, 0)),
            pl.BlockSpec((1, BN, D), lambda i: (1, i, 0)),
            pl.BlockSpec((1, BN, D), lambda i: (0, i, 0)),
            pl.BlockSpec((1, D), lambda i: (0, 0)),
            pl.BlockSpec((1, 1), lambda i: (0, 0)),
            pl.BlockSpec((D, D), lambda i: (0, 0)),
        ],
        out_specs=[
            pl.BlockSpec((BN, D), lambda i: (i, 0)),
            pl.BlockSpec((1, D), lambda i: (0, 0)),
        ],
        out_shape=[
            jax.ShapeDtypeStruct((N, D), jnp.float32),
            jax.ShapeDtypeStruct((1, D), jnp.float32),
        ],
    )(p, p, deg2d, deg2d, b0, alpha, W1)


def _out_body(q0_ref, q1_ref, din_ref, b1_ref, alpha_ref, h2_ref, hg_ref):
    @pl.when(pl.program_id(0) == 0)
    def _():
        hg_ref[...] = jnp.zeros_like(hg_ref)

    nd = lax.rsqrt(jnp.maximum(din_ref[0], 1.0))
    z = (q0_ref[0] + q1_ref[0]) * nd + b1_ref[...]
    a = alpha_ref[0, 0]
    h = jnp.where(z >= 0, z, a * z)
    h2_ref[...] = h
    hg_ref[...] += jnp.sum(h, axis=0, keepdims=True)


@jax.jit
def _out_call(q, deg2d, b1, alpha):
    return pl.pallas_call(
        _out_body,
        grid=(N // BN,),
        in_specs=[
            pl.BlockSpec((1, BN, D), lambda i: (0, i, 0)),
            pl.BlockSpec((1, BN, D), lambda i: (1, i, 0)),
            pl.BlockSpec((1, BN, D), lambda i: (1, i, 0)),
            pl.BlockSpec((1, D), lambda i: (0, 0)),
            pl.BlockSpec((1, 1), lambda i: (0, 0)),
        ],
        out_specs=[
            pl.BlockSpec((BN, D), lambda i: (i, 0)),
            pl.BlockSpec((1, D), lambda i: (0, 0)),
        ],
        out_shape=[
            jax.ShapeDtypeStruct((N, D), jnp.float32),
            jax.ShapeDtypeStruct((1, D), jnp.float32),
        ],
    )(q, q, deg2d, b1, alpha)


def kernel(feat, edge_index, W0, b0, W1, b1, alpha):
    src = edge_index[0]
    dst = edge_index[1]
    parts = _degree_call(src, dst)
    deg2d = _norm_call(parts)
    x0 = _l0_call(feat, W0, deg2d)
    zrows = jnp.zeros((RPT, D), jnp.float32)
    p = _agg_call(x0, src, dst, zrows)
    x1, hg0 = _mid_call(p, deg2d, b0.reshape(1, D), alpha.reshape(1, 1), W1)
    q = _agg_call(x1, src, dst, zrows)
    h2, hg1 = _out_call(q, deg2d, b1.reshape(1, D), alpha.reshape(1, 1))
    hg = jnp.concatenate((hg0, hg1), axis=-1)
    return (h2, hg)


# speedup vs baseline: 12.1574x; 1.1670x over previous
import functools

import jax
import jax.numpy as jnp
from jax import lax
from jax.experimental import pallas as pl
from jax.experimental.pallas import tpu as pltpu
from jax.experimental.pallas import tpu_sc as plsc

N = 10000
E = 320000
D = 128

NC = 2
NS = 16
NW = NC * NS
N_PAD = 10240
EW = E // NW
CH = 80
NCHUNK = EW // CH
SL = N_PAD // NW
RPT = N // NS
BN = 1000

_LANES = 16


def _sc_mesh():
    return plsc.VectorSubcoreMesh(core_axis_name="c", subcore_axis_name="s")


def _degree_body(src_hbm, dst_hbm, parts_hbm, idx_v, hist_v):
    c = lax.axis_index("c")
    s = lax.axis_index("s")
    wid = c * NS + s
    base = wid * EW
    ones = jnp.ones((_LANES,), jnp.float32)
    zeros = jnp.zeros((_LANES,), jnp.float32)
    for a, edges in ((0, src_hbm), (1, dst_hbm)):
        def zero_body(i, carry):
            hist_v[pl.ds(i * _LANES, _LANES)] = zeros
            return carry
        lax.fori_loop(0, N_PAD // _LANES, zero_body, 0)
        pltpu.sync_copy(edges.at[pl.ds(base, EW)], idx_v)

        def acc_body(t, carry):
            idx = idx_v[pl.ds(t * _LANES, _LANES)]
            plsc.addupdate_scatter(hist_v, [idx], ones)
            return carry
        lax.fori_loop(0, EW // _LANES, acc_body, 0)
        pltpu.sync_copy(hist_v, parts_hbm.at[a * NW + wid])


@jax.jit
def _degree_call(src, dst):
    f = functools.partial(
        pl.kernel,
        out_type=jax.ShapeDtypeStruct((2 * NW, N_PAD), jnp.float32),
        mesh=_sc_mesh(),
        scratch_types=[
            pltpu.VMEM((EW,), jnp.int32),
            pltpu.VMEM((N_PAD,), jnp.float32),
        ],
        compiler_params=pltpu.CompilerParams(
            needs_layout_passes=False, use_tc_tiling_on_sc=False),
    )(_degree_body)
    return f(src, dst)


def _norm_body(parts_hbm, deg2d_hbm, pbuf, acc, rowbuf, sem):
    c = lax.axis_index("c")
    s = lax.axis_index("s")
    wid = c * NS + s
    n0 = wid * SL
    zeros = jnp.zeros((_LANES,), jnp.float32)

    def load_body(t, carry):
        pltpu.async_copy(parts_hbm.at[t, pl.ds(n0, SL)], pbuf.at[t], sem)
        return carry
    lax.fori_loop(0, 2 * NW, load_body, 0)

    def wait_body(t, carry):
        pltpu.make_async_copy(parts_hbm.at[t, pl.ds(n0, SL)], pbuf.at[t], sem).wait()
        return carry
    lax.fori_loop(0, 2 * NW, wait_body, 0)

    for a in (0, 1):
        def zero_body(k, carry):
            acc[pl.ds(k * _LANES, _LANES)] = zeros
            return carry
        lax.fori_loop(0, SL // _LANES, zero_body, 0)

        def red_body(i, carry):
            t = i // (SL // _LANES)
            k = i % (SL // _LANES)
            acc[pl.ds(k * _LANES, _LANES)] = (
                acc[pl.ds(k * _LANES, _LANES)]
                + pbuf[a * NW + t, pl.ds(k * _LANES, _LANES)]
            )
            return carry
        lax.fori_loop(0, NW * (SL // _LANES), red_body, 0)

        def bcast_body(g, carry):
            vec16 = acc[pl.ds(g * _LANES, _LANES)]
            for j in range(_LANES):
                vec = lax.broadcast(vec16[j], (_LANES,))
                for k in range(D // _LANES):
                    rowbuf[g * _LANES + j, pl.ds(k * _LANES, _LANES)] = vec
            return carry
        lax.fori_loop(0, SL // _LANES, bcast_body, 0)
        pltpu.sync_copy(rowbuf, deg2d_hbm.at[a, pl.ds(n0, SL)])


@jax.jit
def _norm_call(parts):
    f = functools.partial(
        pl.kernel,
        out_type=jax.ShapeDtypeStruct((2, N_PAD, D), jnp.float32),
        mesh=_sc_mesh(),
        scratch_types=[
            pltpu.VMEM((2 * NW, SL), jnp.float32),
            pltpu.VMEM((SL,), jnp.float32),
            pltpu.VMEM((SL, D), jnp.float32),
            pltpu.SemaphoreType.DMA,
        ],
        compiler_params=pltpu.CompilerParams(use_tc_tiling_on_sc=False),
    )(_norm_body)
    return f(parts)


_NBUF = 3


def _agg_body(x_hbm, src2d_hbm, dst2d_hbm, parts_hbm,
              sidx2d, didx2d, rows0, rows1, rows2, acc_sh,
              gsem0, gsem1, gsem2):
    c = lax.axis_index("c")
    s = lax.axis_index("s")
    wid = c * NS + s
    rows = (rows0, rows1, rows2)
    gsem = (gsem0, gsem1, gsem2)

    zv = jnp.zeros((_LANES,), jnp.float32)

    def zero_body(i, carry):
        rows0[i // (D // _LANES),
              pl.ds((i % (D // _LANES)) * _LANES, _LANES)] = zv
        return carry
    lax.fori_loop(0, CH * (D // _LANES), zero_body, 0)
    for k in range(RPT // CH):
        pltpu.async_copy(rows0, acc_sh.at[pl.ds(s * RPT + k * CH, CH)], gsem0)
    _tail = RPT - (RPT // CH) * CH
    pltpu.async_copy(rows0.at[pl.ds(0, _tail)],
                     acc_sh.at[pl.ds(s * RPT + (RPT // CH) * CH, _tail)],
                     gsem0)
    for k in range(RPT // CH):
        pltpu.make_async_copy(
            rows0, acc_sh.at[pl.ds(s * RPT + k * CH, CH)], gsem0).wait()
    pltpu.make_async_copy(
        rows0.at[pl.ds(0, _tail)],
        acc_sh.at[pl.ds(s * RPT + (RPT // CH) * CH, _tail)], gsem0).wait()

    pltpu.sync_copy(src2d_hbm.at[pl.ds(wid * NCHUNK, NCHUNK)], sidx2d)
    pltpu.sync_copy(dst2d_hbm.at[pl.ds(wid * NCHUNK, NCHUNK)], didx2d)
    plsc.subcore_barrier()

    def start(j, b):
        pltpu.async_copy(x_hbm.at[sidx2d.at[j]], rows[b], gsem[b])

    def wait(b):
        pltpu.make_async_copy(x_hbm.at[sidx2d.at[0]], rows[b], gsem[b]).wait()

    def scatter(j, b):
        pltpu.sync_copy(rows[b], acc_sh.at[didx2d.at[j]], add=True)

    for b in range(_NBUF):
        start(b, b)

    def body(i, carry):
        for b in range(_NBUF):
            j = _NBUF * i + b
            wait(b)
            scatter(j, b)

            @pl.when(j + _NBUF < NCHUNK)
            def _():
                start(j + _NBUF, b)
        return carry
    lax.fori_loop(0, NCHUNK // _NBUF, body, 0)

    for r in range(_NBUF * (NCHUNK // _NBUF), NCHUNK):
        wait(r % _NBUF)
        scatter(r, r % _NBUF)

    plsc.subcore_barrier()
    pltpu.sync_copy(acc_sh.at[pl.ds(s * RPT, RPT)],
                    parts_hbm.at[c, pl.ds(s * RPT, RPT)])


@jax.jit
def _agg_call(x, src, dst):
    f = functools.partial(
        pl.kernel,
        out_type=jax.ShapeDtypeStruct((NC, N, D), jnp.float32),
        mesh=_sc_mesh(),
        scratch_types=[
            pltpu.VMEM((NCHUNK, CH), jnp.int32),
            pltpu.VMEM((NCHUNK, CH), jnp.int32),
            pltpu.VMEM((CH, D), jnp.float32),
            pltpu.VMEM((CH, D), jnp.float32),
            pltpu.VMEM((CH, D), jnp.float32),
            pltpu.VMEM_SHARED((N, D), jnp.float32),
            pltpu.SemaphoreType.DMA,
            pltpu.SemaphoreType.DMA,
            pltpu.SemaphoreType.DMA,
        ],
        compiler_params=pltpu.CompilerParams(use_tc_tiling_on_sc=False),
    )(_agg_body)
    return f(x, src.reshape(E // CH, CH), dst.reshape(E // CH, CH))


def _l0_body(feat_ref, w0_ref, dout_ref, x0_ref):
    ns = lax.rsqrt(jnp.maximum(dout_ref[0], 1.0))
    y = jnp.dot(feat_ref[...], w0_ref[...],
                preferred_element_type=jnp.float32,
                precision=lax.Precision.HIGHEST)
    x0_ref[...] = y * ns


@jax.jit
def _l0_call(feat, W0, deg2d):
    return pl.pallas_call(
        _l0_body,
        grid=(N // BN,),
        in_specs=[
            pl.BlockSpec((BN, D), lambda i: (i, 0)),
            pl.BlockSpec((D, D), lambda i: (0, 0)),
            pl.BlockSpec((1, BN, D), lambda i: (0, i, 0)),
        ],
        out_specs=pl.BlockSpec((BN, D), lambda i: (i, 0)),
        out_shape=jax.ShapeDtypeStruct((N, D), jnp.float32),
    )(feat, W0, deg2d)


def _mid_body(p0_ref, p1_ref, din_ref, dout_ref, b0_ref, alpha_ref, w1_ref,
              x1_ref, hg_ref):
    @pl.when(pl.program_id(0) == 0)
    def _():
        hg_ref[...] = jnp.zeros_like(hg_ref)

    nd = lax.rsqrt(jnp.maximum(din_ref[0], 1.0))
    z = (p0_ref[0] + p1_ref[0]) * nd + b0_ref[...]
    a = alpha_ref[0, 0]
    h = jnp.where(z >= 0, z, a * z)
    hg_ref[...] += jnp.sum(h, axis=0, keepdims=True)
    ns = lax.rsqrt(jnp.maximum(dout_ref[0], 1.0))
    x1_ref[...] = jnp.dot(h, w1_ref[...],
                          preferred_element_type=jnp.float32,
                          precision=lax.Precision.HIGHEST) * ns


@jax.jit
def _mid_call(p, deg2d, b0, alpha, W1):
    return pl.pallas_call(
        _mid_body,
        grid=(N // BN,),
        in_specs=[
            pl.BlockSpec((1, BN, D), lambda i: (0, i, 0)),
            pl.BlockSpec((1, BN, D), lambda i: (1, i, 0)),
            pl.BlockSpec((1, BN, D), lambda i: (1, i, 0)),
            pl.BlockSpec((1, BN, D), lambda i: (0, i, 0)),
            pl.BlockSpec((1, D), lambda i: (0, 0)),
            pl.BlockSpec((1, 1), lambda i: (0, 0)),
            pl.BlockSpec((D, D), lambda i: (0, 0)),
        ],
        out_specs=[
            pl.BlockSpec((BN, D), lambda i: (i, 0)),
            pl.BlockSpec((1, D), lambda i: (0, 0)),
        ],
        out_shape=[
            jax.ShapeDtypeStruct((N, D), jnp.float32),
            jax.ShapeDtypeStruct((1, D), jnp.float32),
        ],
    )(p, p, deg2d, deg2d, b0, alpha, W1)


def _out_body(q0_ref, q1_ref, din_ref, b1_ref, alpha_ref, h2_ref, hg_ref):
    @pl.when(pl.program_id(0) == 0)
    def _():
        hg_ref[...] = jnp.zeros_like(hg_ref)

    nd = lax.rsqrt(jnp.maximum(din_ref[0], 1.0))
    z = (q0_ref[0] + q1_ref[0]) * nd + b1_ref[...]
    a = alpha_ref[0, 0]
    h = jnp.where(z >= 0, z, a * z)
    h2_ref[...] = h
    hg_ref[...] += jnp.sum(h, axis=0, keepdims=True)


@jax.jit
def _out_call(q, deg2d, b1, alpha):
    return pl.pallas_call(
        _out_body,
        grid=(N // BN,),
        in_specs=[
            pl.BlockSpec((1, BN, D), lambda i: (0, i, 0)),
            pl.BlockSpec((1, BN, D), lambda i: (1, i, 0)),
            pl.BlockSpec((1, BN, D), lambda i: (1, i, 0)),
            pl.BlockSpec((1, D), lambda i: (0, 0)),
            pl.BlockSpec((1, 1), lambda i: (0, 0)),
        ],
        out_specs=[
            pl.BlockSpec((BN, D), lambda i: (i, 0)),
            pl.BlockSpec((1, D), lambda i: (0, 0)),
        ],
        out_shape=[
            jax.ShapeDtypeStruct((N, D), jnp.float32),
            jax.ShapeDtypeStruct((1, D), jnp.float32),
        ],
    )(q, q, deg2d, b1, alpha)


def kernel(feat, edge_index, W0, b0, W1, b1, alpha):
    src = edge_index[0]
    dst = edge_index[1]
    parts = _degree_call(src, dst)
    deg2d = _norm_call(parts)
    x0 = _l0_call(feat, W0, deg2d)
    p = _agg_call(x0, src, dst)
    x1, hg0 = _mid_call(p, deg2d, b0.reshape(1, D), alpha.reshape(1, 1), W1)
    q = _agg_call(x1, src, dst)
    h2, hg1 = _out_call(q, deg2d, b1.reshape(1, D), alpha.reshape(1, 1))
    hg = jnp.concatenate((hg0, hg1), axis=-1)
    return (h2, hg)


# speedup vs baseline: 12.8154x; 1.0541x over previous
import functools

import jax
import jax.numpy as jnp
from jax import lax
from jax.experimental import pallas as pl
from jax.experimental.pallas import tpu as pltpu
from jax.experimental.pallas import tpu_sc as plsc

N = 10000
E = 320000
D = 128

NC = 2
NS = 16
NW = NC * NS
N_PAD = 10240
EW = E // NW
CH = 80
NCHUNK = EW // CH
SL = N_PAD // NW
RPT = N // NS
BN = 2000

_LANES = 16


def _sc_mesh():
    return plsc.VectorSubcoreMesh(core_axis_name="c", subcore_axis_name="s")


def _degree_body(src_hbm, dst_hbm, parts_hbm, idx_v, hist_v):
    c = lax.axis_index("c")
    s = lax.axis_index("s")
    wid = c * NS + s
    base = wid * EW
    ones = jnp.ones((_LANES,), jnp.float32)
    zeros = jnp.zeros((_LANES,), jnp.float32)
    for a, edges in ((0, src_hbm), (1, dst_hbm)):
        def zero_body(i, carry):
            hist_v[pl.ds(i * _LANES, _LANES)] = zeros
            return carry
        lax.fori_loop(0, N_PAD // _LANES, zero_body, 0)
        pltpu.sync_copy(edges.at[pl.ds(base, EW)], idx_v)

        def acc_body(t, carry):
            idx = idx_v[pl.ds(t * _LANES, _LANES)]
            plsc.addupdate_scatter(hist_v, [idx], ones)
            return carry
        lax.fori_loop(0, EW // _LANES, acc_body, 0)
        pltpu.sync_copy(hist_v, parts_hbm.at[a * NW + wid])


@jax.jit
def _degree_call(src, dst):
    f = functools.partial(
        pl.kernel,
        out_type=jax.ShapeDtypeStruct((2 * NW, N_PAD), jnp.float32),
        mesh=_sc_mesh(),
        scratch_types=[
            pltpu.VMEM((EW,), jnp.int32),
            pltpu.VMEM((N_PAD,), jnp.float32),
        ],
        compiler_params=pltpu.CompilerParams(
            needs_layout_passes=False, use_tc_tiling_on_sc=False),
    )(_degree_body)
    return f(src, dst)


def _norm_body(parts_hbm, deg2d_hbm, pbuf, acc, rowbuf, sem):
    c = lax.axis_index("c")
    s = lax.axis_index("s")
    wid = c * NS + s
    n0 = wid * SL
    zeros = jnp.zeros((_LANES,), jnp.float32)

    def load_body(t, carry):
        pltpu.async_copy(parts_hbm.at[t, pl.ds(n0, SL)], pbuf.at[t], sem)
        return carry
    lax.fori_loop(0, 2 * NW, load_body, 0)

    def wait_body(t, carry):
        pltpu.make_async_copy(parts_hbm.at[t, pl.ds(n0, SL)], pbuf.at[t], sem).wait()
        return carry
    lax.fori_loop(0, 2 * NW, wait_body, 0)

    for a in (0, 1):
        def zero_body(k, carry):
            acc[pl.ds(k * _LANES, _LANES)] = zeros
            return carry
        lax.fori_loop(0, SL // _LANES, zero_body, 0)

        def red_body(i, carry):
            t = i // (SL // _LANES)
            k = i % (SL // _LANES)
            acc[pl.ds(k * _LANES, _LANES)] = (
                acc[pl.ds(k * _LANES, _LANES)]
                + pbuf[a * NW + t, pl.ds(k * _LANES, _LANES)]
            )
            return carry
        lax.fori_loop(0, NW * (SL // _LANES), red_body, 0)

        def bcast_body(g, carry):
            vec16 = acc[pl.ds(g * _LANES, _LANES)]
            for j in range(_LANES):
                vec = lax.broadcast(vec16[j], (_LANES,))
                for k in range(D // _LANES):
                    rowbuf[g * _LANES + j, pl.ds(k * _LANES, _LANES)] = vec
            return carry
        lax.fori_loop(0, SL // _LANES, bcast_body, 0)
        pltpu.sync_copy(rowbuf, deg2d_hbm.at[a, pl.ds(n0, SL)])


@jax.jit
def _norm_call(parts):
    f = functools.partial(
        pl.kernel,
        out_type=jax.ShapeDtypeStruct((2, N_PAD, D), jnp.float32),
        mesh=_sc_mesh(),
        scratch_types=[
            pltpu.VMEM((2 * NW, SL), jnp.float32),
            pltpu.VMEM((SL,), jnp.float32),
            pltpu.VMEM((SL, D), jnp.float32),
            pltpu.SemaphoreType.DMA,
        ],
        compiler_params=pltpu.CompilerParams(use_tc_tiling_on_sc=False),
    )(_norm_body)
    return f(parts)


_NBUF = 3


def _agg_body(x_hbm, src2d_hbm, dst2d_hbm, parts_hbm,
              sidx2d, didx2d, rows0, rows1, rows2, acc_sh,
              gsem0, gsem1, gsem2):
    c = lax.axis_index("c")
    s = lax.axis_index("s")
    wid = c * NS + s
    rows = (rows0, rows1, rows2)
    gsem = (gsem0, gsem1, gsem2)

    pltpu.async_copy(src2d_hbm.at[pl.ds(wid * NCHUNK, NCHUNK)], sidx2d, gsem2)
    pltpu.async_copy(dst2d_hbm.at[pl.ds(wid * NCHUNK, NCHUNK)], didx2d, gsem2)

    zv = jnp.zeros((_LANES,), jnp.float32)

    def zero_body(i, carry):
        rows1[i // (D // _LANES),
              pl.ds((i % (D // _LANES)) * _LANES, _LANES)] = zv
        return carry
    lax.fori_loop(0, CH * (D // _LANES), zero_body, 0)
    _tail = RPT - (RPT // CH) * CH
    for k in range(RPT // CH):
        pltpu.async_copy(rows1, acc_sh.at[pl.ds(s * RPT + k * CH, CH)], gsem1)
    pltpu.async_copy(rows1.at[pl.ds(0, _tail)],
                     acc_sh.at[pl.ds(s * RPT + (RPT // CH) * CH, _tail)],
                     gsem1)

    def start(j, b):
        pltpu.async_copy(x_hbm.at[sidx2d.at[j]], rows[b], gsem[b])

    def wait(b):
        pltpu.make_async_copy(x_hbm.at[sidx2d.at[0]], rows[b], gsem[b]).wait()

    def scatter(j, b):
        pltpu.sync_copy(rows[b], acc_sh.at[didx2d.at[j]], add=True)

    pltpu.make_async_copy(
        src2d_hbm.at[pl.ds(wid * NCHUNK, NCHUNK)], sidx2d, gsem2).wait()
    pltpu.make_async_copy(
        dst2d_hbm.at[pl.ds(wid * NCHUNK, NCHUNK)], didx2d, gsem2).wait()
    start(0, 0)
    start(2, 2)
    for k in range(RPT // CH):
        pltpu.make_async_copy(
            rows1, acc_sh.at[pl.ds(s * RPT + k * CH, CH)], gsem1).wait()
    pltpu.make_async_copy(
        rows1.at[pl.ds(0, _tail)],
        acc_sh.at[pl.ds(s * RPT + (RPT // CH) * CH, _tail)], gsem1).wait()
    start(1, 1)
    plsc.subcore_barrier()

    def body(i, carry):
        for b in range(_NBUF):
            j = _NBUF * i + b
            wait(b)
            scatter(j, b)

            @pl.when(j + _NBUF < NCHUNK)
            def _():
                start(j + _NBUF, b)
        return carry
    lax.fori_loop(0, NCHUNK // _NBUF, body, 0)

    for r in range(_NBUF * (NCHUNK // _NBUF), NCHUNK):
        wait(r % _NBUF)
        scatter(r, r % _NBUF)

    plsc.subcore_barrier()
    pltpu.sync_copy(acc_sh.at[pl.ds(s * RPT, RPT)],
                    parts_hbm.at[c, pl.ds(s * RPT, RPT)])


@jax.jit
def _agg_call(x, src, dst):
    f = functools.partial(
        pl.kernel,
        out_type=jax.ShapeDtypeStruct((NC, N, D), jnp.float32),
        mesh=_sc_mesh(),
        scratch_types=[
            pltpu.VMEM((NCHUNK, CH), jnp.int32),
            pltpu.VMEM((NCHUNK, CH), jnp.int32),
            pltpu.VMEM((CH, D), jnp.float32),
            pltpu.VMEM((CH, D), jnp.float32),
            pltpu.VMEM((CH, D), jnp.float32),
            pltpu.VMEM_SHARED((N, D), jnp.float32),
            pltpu.SemaphoreType.DMA,
            pltpu.SemaphoreType.DMA,
            pltpu.SemaphoreType.DMA,
        ],
        compiler_params=pltpu.CompilerParams(use_tc_tiling_on_sc=False),
    )(_agg_body)
    return f(x, src.reshape(E // CH, CH), dst.reshape(E // CH, CH))


def _l0_body(feat_ref, w0_ref, dout_ref, x0_ref):
    ns = lax.rsqrt(jnp.maximum(dout_ref[0], 1.0))
    y = jnp.dot(feat_ref[...], w0_ref[...],
                preferred_element_type=jnp.float32,
                precision=lax.Precision.HIGHEST)
    x0_ref[...] = y * ns


@jax.jit
def _l0_call(feat, W0, deg2d):
    return pl.pallas_call(
        _l0_body,
        grid=(N // BN,),
        in_specs=[
            pl.BlockSpec((BN, D), lambda i: (i, 0)),
            pl.BlockSpec((D, D), lambda i: (0, 0)),
            pl.BlockSpec((1, BN, D), lambda i: (0, i, 0)),
        ],
        out_specs=pl.BlockSpec((BN, D), lambda i: (i, 0)),
        out_shape=jax.ShapeDtypeStruct((N, D), jnp.float32),
    )(feat, W0, deg2d)


def _mid_body(p0_ref, p1_ref, din_ref, dout_ref, b0_ref, alpha_ref, w1_ref,
              x1_ref, hg_ref):
    @pl.when(pl.program_id(0) == 0)
    def _():
        hg_ref[...] = jnp.zeros_like(hg_ref)

    nd = lax.rsqrt(jnp.maximum(din_ref[0], 1.0))
    z = (p0_ref[0] + p1_ref[0]) * nd + b0_ref[...]
    a = alpha_ref[0, 0]
    h = jnp.where(z >= 0, z, a * z)
    hg_ref[...] += jnp.sum(h, axis=0, keepdims=True)
    ns = lax.rsqrt(jnp.maximum(dout_ref[0], 1.0))
    x1_ref[...] = jnp.dot(h, w1_ref[...],
                          preferred_element_type=jnp.float32,
                          precision=lax.Precision.HIGHEST) * ns


@jax.jit
def _mid_call(p, deg2d, b0, alpha, W1):
    return pl.pallas_call(
        _mid_body,
        grid=(N // BN,),
        in_specs=[
            pl.BlockSpec((1, BN, D), lambda i: (0, i, 0)),
            pl.BlockSpec((1, BN, D), lambda i: (1, i, 0)),
            pl.BlockSpec((1, BN, D), lambda i: (1, i, 0)),
            pl.BlockSpec((1, BN, D), lambda i: (0, i, 0)),
            pl.BlockSpec((1, D), lambda i: (0, 0)),
            pl.BlockSpec((1, 1), lambda i: (0, 0)),
            pl.BlockSpec((D, D), lambda i: (0, 0)),
        ],
        out_specs=[
            pl.BlockSpec((BN, D), lambda i: (i, 0)),
            pl.BlockSpec((1, D), lambda i: (0, 0)),
        ],
        out_shape=[
            jax.ShapeDtypeStruct((N, D), jnp.float32),
            jax.ShapeDtypeStruct((1, D), jnp.float32),
        ],
    )(p, p, deg2d, deg2d, b0, alpha, W1)


def _out_body(q0_ref, q1_ref, din_ref, b1_ref, alpha_ref, h2_ref, hg_ref):
    @pl.when(pl.program_id(0) == 0)
    def _():
        hg_ref[...] = jnp.zeros_like(hg_ref)

    nd = lax.rsqrt(jnp.maximum(din_ref[0], 1.0))
    z = (q0_ref[0] + q1_ref[0]) * nd + b1_ref[...]
    a = alpha_ref[0, 0]
    h = jnp.where(z >= 0, z, a * z)
    h2_ref[...] = h
    hg_ref[...] += jnp.sum(h, axis=0, keepdims=True)


@jax.jit
def _out_call(q, deg2d, b1, alpha):
    return pl.pallas_call(
        _out_body,
        grid=(N // BN,),
        in_specs=[
            pl.BlockSpec((1, BN, D), lambda i: (0, i, 0)),
            pl.BlockSpec((1, BN, D), lambda i: (1, i, 0)),
            pl.BlockSpec((1, BN, D), lambda i: (1, i, 0)),
            pl.BlockSpec((1, D), lambda i: (0, 0)),
            pl.BlockSpec((1, 1), lambda i: (0, 0)),
        ],
        out_specs=[
            pl.BlockSpec((BN, D), lambda i: (i, 0)),
            pl.BlockSpec((1, D), lambda i: (0, 0)),
        ],
        out_shape=[
            jax.ShapeDtypeStruct((N, D), jnp.float32),
            jax.ShapeDtypeStruct((1, D), jnp.float32),
        ],
    )(q, q, deg2d, b1, alpha)


def kernel(feat, edge_index, W0, b0, W1, b1, alpha):
    src = edge_index[0]
    dst = edge_index[1]
    parts = _degree_call(src, dst)
    deg2d = _norm_call(parts)
    x0 = _l0_call(feat, W0, deg2d)
    p = _agg_call(x0, src, dst)
    x1, hg0 = _mid_call(p, deg2d, b0.reshape(1, D), alpha.reshape(1, 1), W1)
    q = _agg_call(x1, src, dst)
    h2, hg1 = _out_call(q, deg2d, b1.reshape(1, D), alpha.reshape(1, 1))
    hg = jnp.concatenate((hg0, hg1), axis=-1)
    return (h2, hg)


# speedup vs baseline: 13.1181x; 1.0236x over previous
import functools

import jax
import jax.numpy as jnp
from jax import lax
from jax.experimental import pallas as pl
from jax.experimental.pallas import tpu as pltpu
from jax.experimental.pallas import tpu_sc as plsc

N = 10000
E = 320000
D = 128

NC = 2
NS = 16
NW = NC * NS
N_PAD = 10240
EW = E // NW
CH = 80
NCHUNK = EW // CH
SL = N_PAD // NW
RPT = N // NS
BN = 2000

_LANES = 16


def _sc_mesh():
    return plsc.VectorSubcoreMesh(core_axis_name="c", subcore_axis_name="s")


def _degree_body(src_hbm, dst_hbm, parts_hbm, sidx_v, didx_v,
                 hout_v, hin_v, sem):
    c = lax.axis_index("c")
    s = lax.axis_index("s")
    wid = c * NS + s
    base = wid * EW
    ones = jnp.ones((_LANES,), jnp.float32)
    zeros = jnp.zeros((_LANES,), jnp.float32)

    pltpu.async_copy(src_hbm.at[pl.ds(base, EW)], sidx_v, sem)
    pltpu.async_copy(dst_hbm.at[pl.ds(base, EW)], didx_v, sem)

    def zero_body(i, carry):
        hout_v[pl.ds(i * _LANES, _LANES)] = zeros
        hin_v[pl.ds(i * _LANES, _LANES)] = zeros
        return carry
    lax.fori_loop(0, N_PAD // _LANES, zero_body, 0)

    pltpu.make_async_copy(src_hbm.at[pl.ds(base, EW)], sidx_v, sem).wait()
    pltpu.make_async_copy(dst_hbm.at[pl.ds(base, EW)], didx_v, sem).wait()

    def acc_body(t, carry):
        plsc.addupdate_scatter(hout_v, [sidx_v[pl.ds(t * _LANES, _LANES)]],
                               ones)
        plsc.addupdate_scatter(hin_v, [didx_v[pl.ds(t * _LANES, _LANES)]],
                               ones)
        return carry
    lax.fori_loop(0, EW // _LANES, acc_body, 0)
    pltpu.sync_copy(hout_v, parts_hbm.at[wid])
    pltpu.sync_copy(hin_v, parts_hbm.at[NW + wid])


@jax.jit
def _degree_call(src, dst):
    f = functools.partial(
        pl.kernel,
        out_type=jax.ShapeDtypeStruct((2 * NW, N_PAD), jnp.float32),
        mesh=_sc_mesh(),
        scratch_types=[
            pltpu.VMEM((EW,), jnp.int32),
            pltpu.VMEM((EW,), jnp.int32),
            pltpu.VMEM((N_PAD,), jnp.float32),
            pltpu.VMEM((N_PAD,), jnp.float32),
            pltpu.SemaphoreType.DMA,
        ],
        compiler_params=pltpu.CompilerParams(
            needs_layout_passes=False, use_tc_tiling_on_sc=False),
    )(_degree_body)
    return f(src, dst)


def _norm_body(parts_hbm, deg2d_hbm, pbuf, acc, rowbuf, sem):
    c = lax.axis_index("c")
    s = lax.axis_index("s")
    wid = c * NS + s
    n0 = wid * SL
    zeros = jnp.zeros((_LANES,), jnp.float32)

    def load_body(t, carry):
        pltpu.async_copy(parts_hbm.at[t, pl.ds(n0, SL)], pbuf.at[t], sem)
        return carry
    lax.fori_loop(0, 2 * NW, load_body, 0)

    def wait_body(t, carry):
        pltpu.make_async_copy(parts_hbm.at[t, pl.ds(n0, SL)], pbuf.at[t], sem).wait()
        return carry
    lax.fori_loop(0, 2 * NW, wait_body, 0)

    for a in (0, 1):
        def zero_body(k, carry):
            acc[pl.ds(k * _LANES, _LANES)] = zeros
            return carry
        lax.fori_loop(0, SL // _LANES, zero_body, 0)

        def red_body(i, carry):
            t = i // (SL // _LANES)
            k = i % (SL // _LANES)
            acc[pl.ds(k * _LANES, _LANES)] = (
                acc[pl.ds(k * _LANES, _LANES)]
                + pbuf[a * NW + t, pl.ds(k * _LANES, _LANES)]
            )
            return carry
        lax.fori_loop(0, NW * (SL // _LANES), red_body, 0)

        def bcast_body(g, carry):
            vec16 = acc[pl.ds(g * _LANES, _LANES)]
            for j in range(_LANES):
                rowbuf[g * _LANES + j, pl.ds(0, _LANES)] = lax.broadcast(
                    vec16[j], (_LANES,))
            return carry
        lax.fori_loop(0, SL // _LANES, bcast_body, 0)
        pltpu.sync_copy(rowbuf, deg2d_hbm.at[a, pl.ds(n0, SL)])


@jax.jit
def _norm_call(parts):
    f = functools.partial(
        pl.kernel,
        out_type=jax.ShapeDtypeStruct((2, N_PAD, _LANES), jnp.float32),
        mesh=_sc_mesh(),
        scratch_types=[
            pltpu.VMEM((2 * NW, SL), jnp.float32),
            pltpu.VMEM((SL,), jnp.float32),
            pltpu.VMEM((SL, _LANES), jnp.float32),
            pltpu.SemaphoreType.DMA,
        ],
        compiler_params=pltpu.CompilerParams(use_tc_tiling_on_sc=False),
    )(_norm_body)
    return f(parts)


_NBUF = 3


def _agg_body(x_hbm, src2d_hbm, dst2d_hbm, parts_hbm,
              sidx2d, didx2d, rows0, rows1, rows2, acc_sh,
              gsem0, gsem1, gsem2):
    c = lax.axis_index("c")
    s = lax.axis_index("s")
    wid = c * NS + s
    rows = (rows0, rows1, rows2)
    gsem = (gsem0, gsem1, gsem2)

    pltpu.async_copy(src2d_hbm.at[pl.ds(wid * NCHUNK, NCHUNK)], sidx2d, gsem2)
    pltpu.async_copy(dst2d_hbm.at[pl.ds(wid * NCHUNK, NCHUNK)], didx2d, gsem2)

    zv = jnp.zeros((_LANES,), jnp.float32)

    def zero_body(i, carry):
        rows1[i // (D // _LANES),
              pl.ds((i % (D // _LANES)) * _LANES, _LANES)] = zv
        return carry
    lax.fori_loop(0, CH * (D // _LANES), zero_body, 0)
    _tail = RPT - (RPT // CH) * CH
    for k in range(RPT // CH):
        pltpu.async_copy(rows1, acc_sh.at[pl.ds(s * RPT + k * CH, CH)], gsem1)
    pltpu.async_copy(rows1.at[pl.ds(0, _tail)],
                     acc_sh.at[pl.ds(s * RPT + (RPT // CH) * CH, _tail)],
                     gsem1)

    def start(j, b):
        pltpu.async_copy(x_hbm.at[sidx2d.at[j]], rows[b], gsem[b])

    def wait(b):
        pltpu.make_async_copy(x_hbm.at[sidx2d.at[0]], rows[b], gsem[b]).wait()

    def scatter(j, b):
        pltpu.sync_copy(rows[b], acc_sh.at[didx2d.at[j]], add=True)

    pltpu.make_async_copy(
        src2d_hbm.at[pl.ds(wid * NCHUNK, NCHUNK)], sidx2d, gsem2).wait()
    pltpu.make_async_copy(
        dst2d_hbm.at[pl.ds(wid * NCHUNK, NCHUNK)], didx2d, gsem2).wait()
    start(0, 0)
    start(2, 2)
    for k in range(RPT // CH):
        pltpu.make_async_copy(
            rows1, acc_sh.at[pl.ds(s * RPT + k * CH, CH)], gsem1).wait()
    pltpu.make_async_copy(
        rows1.at[pl.ds(0, _tail)],
        acc_sh.at[pl.ds(s * RPT + (RPT // CH) * CH, _tail)], gsem1).wait()
    start(1, 1)
    plsc.subcore_barrier()

    def body(i, carry):
        for b in range(_NBUF):
            j = _NBUF * i + b
            wait(b)
            scatter(j, b)

            @pl.when(j + _NBUF < NCHUNK)
            def _():
                start(j + _NBUF, b)
        return carry
    lax.fori_loop(0, NCHUNK // _NBUF, body, 0)

    for r in range(_NBUF * (NCHUNK // _NBUF), NCHUNK):
        wait(r % _NBUF)
        scatter(r, r % _NBUF)

    plsc.subcore_barrier()
    pltpu.sync_copy(acc_sh.at[pl.ds(s * RPT, RPT)],
                    parts_hbm.at[c, pl.ds(s * RPT, RPT)])


@jax.jit
def _agg_call(x, src, dst):
    f = functools.partial(
        pl.kernel,
        out_type=jax.ShapeDtypeStruct((NC, N, D), jnp.float32),
        mesh=_sc_mesh(),
        scratch_types=[
            pltpu.VMEM((NCHUNK, CH), jnp.int32),
            pltpu.VMEM((NCHUNK, CH), jnp.int32),
            pltpu.VMEM((CH, D), jnp.float32),
            pltpu.VMEM((CH, D), jnp.float32),
            pltpu.VMEM((CH, D), jnp.float32),
            pltpu.VMEM_SHARED((N, D), jnp.float32),
            pltpu.SemaphoreType.DMA,
            pltpu.SemaphoreType.DMA,
            pltpu.SemaphoreType.DMA,
        ],
        compiler_params=pltpu.CompilerParams(use_tc_tiling_on_sc=False),
    )(_agg_body)
    return f(x, src.reshape(E // CH, CH), dst.reshape(E // CH, CH))


def _l0_body(feat_ref, w0_ref, dout_ref, x0_ref):
    ns = lax.rsqrt(jnp.maximum(dout_ref[0][:, :1], 1.0))
    y = jnp.dot(feat_ref[...], w0_ref[...],
                preferred_element_type=jnp.float32,
                precision=lax.Precision.HIGHEST)
    x0_ref[...] = y * ns


@jax.jit
def _l0_call(feat, W0, deg2d):
    return pl.pallas_call(
        _l0_body,
        grid=(N // BN,),
        in_specs=[
            pl.BlockSpec((BN, D), lambda i: (i, 0)),
            pl.BlockSpec((D, D), lambda i: (0, 0)),
            pl.BlockSpec((1, BN, _LANES), lambda i: (0, i, 0)),
        ],
        out_specs=pl.BlockSpec((BN, D), lambda i: (i, 0)),
        out_shape=jax.ShapeDtypeStruct((N, D), jnp.float32),
    )(feat, W0, deg2d)


def _mid_body(p0_ref, p1_ref, din_ref, dout_ref, b0_ref, alpha_ref, w1_ref,
              x1_ref, hg_ref):
    @pl.when(pl.program_id(0) == 0)
    def _():
        hg_ref[...] = jnp.zeros_like(hg_ref)

    nd = lax.rsqrt(jnp.maximum(din_ref[0][:, :1], 1.0))
    z = (p0_ref[0] + p1_ref[0]) * nd + b0_ref[...]
    a = alpha_ref[0, 0]
    h = jnp.where(z >= 0, z, a * z)
    hg_ref[...] += jnp.sum(h, axis=0, keepdims=True)
    ns = lax.rsqrt(jnp.maximum(dout_ref[0][:, :1], 1.0))
    x1_ref[...] = jnp.dot(h, w1_ref[...],
                          preferred_element_type=jnp.float32,
                          precision=lax.Precision.HIGHEST) * ns


@jax.jit
def _mid_call(p, deg2d, b0, alpha, W1):
    return pl.pallas_call(
        _mid_body,
        grid=(N // BN,),
        in_specs=[
            pl.BlockSpec((1, BN, D), lambda i: (0, i, 0)),
            pl.BlockSpec((1, BN, D), lambda i: (1, i, 0)),
            pl.BlockSpec((1, BN, _LANES), lambda i: (1, i, 0)),
            pl.BlockSpec((1, BN, _LANES), lambda i: (0, i, 0)),
            pl.BlockSpec((1, D), lambda i: (0, 0)),
            pl.BlockSpec((1, 1), lambda i: (0, 0)),
            pl.BlockSpec((D, D), lambda i: (0, 0)),
        ],
        out_specs=[
            pl.BlockSpec((BN, D), lambda i: (i, 0)),
            pl.BlockSpec((1, D), lambda i: (0, 0)),
        ],
        out_shape=[
            jax.ShapeDtypeStruct((N, D), jnp.float32),
            jax.ShapeDtypeStruct((1, D), jnp.float32),
        ],
    )(p, p, deg2d, deg2d, b0, alpha, W1)


def _out_body(q0_ref, q1_ref, din_ref, b1_ref, alpha_ref, h2_ref, hg_ref):
    @pl.when(pl.program_id(0) == 0)
    def _():
        hg_ref[...] = jnp.zeros_like(hg_ref)

    nd = lax.rsqrt(jnp.maximum(din_ref[0][:, :1], 1.0))
    z = (q0_ref[0] + q1_ref[0]) * nd + b1_ref[...]
    a = alpha_ref[0, 0]
    h = jnp.where(z >= 0, z, a * z)
    h2_ref[...] = h
    hg_ref[...] += jnp.sum(h, axis=0, keepdims=True)


@jax.jit
def _out_call(q, deg2d, b1, alpha):
    return pl.pallas_call(
        _out_body,
        grid=(N // BN,),
        in_specs=[
            pl.BlockSpec((1, BN, D), lambda i: (0, i, 0)),
            pl.BlockSpec((1, BN, D), lambda i: (1, i, 0)),
            pl.BlockSpec((1, BN, _LANES), lambda i: (1, i, 0)),
            pl.BlockSpec((1, D), lambda i: (0, 0)),
            pl.BlockSpec((1, 1), lambda i: (0, 0)),
        ],
        out_specs=[
            pl.BlockSpec((BN, D), lambda i: (i, 0)),
            pl.BlockSpec((1, D), lambda i: (0, 0)),
        ],
        out_shape=[
            jax.ShapeDtypeStruct((N, D), jnp.float32),
            jax.ShapeDtypeStruct((1, D), jnp.float32),
        ],
    )(q, q, deg2d, b1, alpha)


def kernel(feat, edge_index, W0, b0, W1, b1, alpha):
    src = edge_index[0]
    dst = edge_index[1]
    parts = _degree_call(src, dst)
    deg2d = _norm_call(parts)
    x0 = _l0_call(feat, W0, deg2d)
    p = _agg_call(x0, src, dst)
    x1, hg0 = _mid_call(p, deg2d, b0.reshape(1, D), alpha.reshape(1, 1), W1)
    q = _agg_call(x1, src, dst)
    h2, hg1 = _out_call(q, deg2d, b1.reshape(1, D), alpha.reshape(1, 1))
    hg = jnp.concatenate((hg0, hg1), axis=-1)
    return (h2, hg)


# speedup vs baseline: 13.7376x; 1.0472x over previous
import functools

import jax
import jax.numpy as jnp
from jax import lax
from jax.experimental import pallas as pl
from jax.experimental.pallas import tpu as pltpu
from jax.experimental.pallas import tpu_sc as plsc

N = 10000
E = 320000
D = 128

NC = 2
NS = 16
NW = NC * NS
N_PAD = 10240
EW = E // NW
CH = 80
NCHUNK = EW // CH
SL = N_PAD // NW
RPT = N // NS
BN = 2000

_LANES = 16


def _sc_mesh():
    return plsc.VectorSubcoreMesh(core_axis_name="c", subcore_axis_name="s")


def _degree_body(ei_hbm, parts_hbm, sidx2, didx2, hout_v, hin_v, sem):
    c = lax.axis_index("c")
    s = lax.axis_index("s")
    wid = c * NS + s
    ones = jnp.ones((_LANES,), jnp.float32)
    zeros = jnp.zeros((_LANES,), jnp.float32)

    pltpu.async_copy(ei_hbm.at[pl.ds(wid * NCHUNK, NCHUNK)], sidx2, sem)
    pltpu.async_copy(ei_hbm.at[pl.ds(E // CH + wid * NCHUNK, NCHUNK)],
                     didx2, sem)

    def zero_body(i, carry):
        hout_v[pl.ds(i * _LANES, _LANES)] = zeros
        hin_v[pl.ds(i * _LANES, _LANES)] = zeros
        return carry
    lax.fori_loop(0, N_PAD // _LANES, zero_body, 0)

    pltpu.make_async_copy(ei_hbm.at[pl.ds(wid * NCHUNK, NCHUNK)],
                          sidx2, sem).wait()
    pltpu.make_async_copy(ei_hbm.at[pl.ds(E // CH + wid * NCHUNK, NCHUNK)],
                          didx2, sem).wait()

    def acc_body(t, carry):
        r = t // (CH // _LANES)
        k = t % (CH // _LANES)
        plsc.addupdate_scatter(hout_v, [sidx2[r, pl.ds(k * _LANES, _LANES)]],
                               ones)
        plsc.addupdate_scatter(hin_v, [didx2[r, pl.ds(k * _LANES, _LANES)]],
                               ones)
        return carry
    lax.fori_loop(0, EW // _LANES, acc_body, 0)
    pltpu.sync_copy(hout_v, parts_hbm.at[wid])
    pltpu.sync_copy(hin_v, parts_hbm.at[NW + wid])


@jax.jit
def _degree_call(ei2d):
    f = functools.partial(
        pl.kernel,
        out_type=jax.ShapeDtypeStruct((2 * NW, N_PAD), jnp.float32),
        mesh=_sc_mesh(),
        scratch_types=[
            pltpu.VMEM((NCHUNK, CH), jnp.int32),
            pltpu.VMEM((NCHUNK, CH), jnp.int32),
            pltpu.VMEM((N_PAD,), jnp.float32),
            pltpu.VMEM((N_PAD,), jnp.float32),
            pltpu.SemaphoreType.DMA,
        ],
        compiler_params=pltpu.CompilerParams(
            needs_layout_passes=False, use_tc_tiling_on_sc=False),
    )(_degree_body)
    return f(ei2d)


def _norm_body(parts_hbm, deg2d_hbm, pbuf, acc, rowbuf, sem):
    c = lax.axis_index("c")
    s = lax.axis_index("s")
    wid = c * NS + s
    n0 = wid * SL
    zeros = jnp.zeros((_LANES,), jnp.float32)

    def load_body(t, carry):
        pltpu.async_copy(parts_hbm.at[t, pl.ds(n0, SL)], pbuf.at[t], sem)
        return carry
    lax.fori_loop(0, 2 * NW, load_body, 0)

    def wait_body(t, carry):
        pltpu.make_async_copy(parts_hbm.at[t, pl.ds(n0, SL)], pbuf.at[t], sem).wait()
        return carry
    lax.fori_loop(0, 2 * NW, wait_body, 0)

    for a in (0, 1):
        def zero_body(k, carry):
            acc[pl.ds(k * _LANES, _LANES)] = zeros
            return carry
        lax.fori_loop(0, SL // _LANES, zero_body, 0)

        def red_body(i, carry):
            t = i // (SL // _LANES)
            k = i % (SL // _LANES)
            acc[pl.ds(k * _LANES, _LANES)] = (
                acc[pl.ds(k * _LANES, _LANES)]
                + pbuf[a * NW + t, pl.ds(k * _LANES, _LANES)]
            )
            return carry
        lax.fori_loop(0, NW * (SL // _LANES), red_body, 0)

        def bcast_body(g, carry):
            vec16 = acc[pl.ds(g * _LANES, _LANES)]
            for j in range(_LANES):
                rowbuf[g * _LANES + j, pl.ds(0, _LANES)] = lax.broadcast(
                    vec16[j], (_LANES,))
            return carry
        lax.fori_loop(0, SL // _LANES, bcast_body, 0)
        pltpu.sync_copy(rowbuf, deg2d_hbm.at[a, pl.ds(n0, SL)])


@jax.jit
def _norm_call(parts):
    f = functools.partial(
        pl.kernel,
        out_type=jax.ShapeDtypeStruct((2, N_PAD, _LANES), jnp.float32),
        mesh=_sc_mesh(),
        scratch_types=[
            pltpu.VMEM((2 * NW, SL), jnp.float32),
            pltpu.VMEM((SL,), jnp.float32),
            pltpu.VMEM((SL, _LANES), jnp.float32),
            pltpu.SemaphoreType.DMA,
        ],
        compiler_params=pltpu.CompilerParams(use_tc_tiling_on_sc=False),
    )(_norm_body)
    return f(parts)


_NBUF = 3


def _agg_body(x_hbm, ei_hbm, parts_hbm,
              sidx2d, didx2d, rows0, rows1, rows2, acc_sh,
              gsem0, gsem1, gsem2):
    c = lax.axis_index("c")
    s = lax.axis_index("s")
    wid = c * NS + s
    rows = (rows0, rows1, rows2)
    gsem = (gsem0, gsem1, gsem2)

    pltpu.async_copy(ei_hbm.at[pl.ds(wid * NCHUNK, NCHUNK)], sidx2d, gsem2)
    pltpu.async_copy(ei_hbm.at[pl.ds(E // CH + wid * NCHUNK, NCHUNK)],
                     didx2d, gsem2)

    zv = jnp.zeros((_LANES,), jnp.float32)

    def zero_body(i, carry):
        rows1[i // (D // _LANES),
              pl.ds((i % (D // _LANES)) * _LANES, _LANES)] = zv
        return carry
    lax.fori_loop(0, CH * (D // _LANES), zero_body, 0)
    _tail = RPT - (RPT // CH) * CH
    for k in range(RPT // CH):
        pltpu.async_copy(rows1, acc_sh.at[pl.ds(s * RPT + k * CH, CH)], gsem1)
    pltpu.async_copy(rows1.at[pl.ds(0, _tail)],
                     acc_sh.at[pl.ds(s * RPT + (RPT // CH) * CH, _tail)],
                     gsem1)

    def start(j, b):
        pltpu.async_copy(x_hbm.at[sidx2d.at[j]], rows[b], gsem[b])

    def wait(b):
        pltpu.make_async_copy(x_hbm.at[sidx2d.at[0]], rows[b], gsem[b]).wait()

    def scatter(j, b):
        pltpu.sync_copy(rows[b], acc_sh.at[didx2d.at[j]], add=True)

    pltpu.make_async_copy(
        ei_hbm.at[pl.ds(wid * NCHUNK, NCHUNK)], sidx2d, gsem2).wait()
    pltpu.make_async_copy(
        ei_hbm.at[pl.ds(E // CH + wid * NCHUNK, NCHUNK)], didx2d, gsem2).wait()
    start(0, 0)
    start(2, 2)
    for k in range(RPT // CH):
        pltpu.make_async_copy(
            rows1, acc_sh.at[pl.ds(s * RPT + k * CH, CH)], gsem1).wait()
    pltpu.make_async_copy(
        rows1.at[pl.ds(0, _tail)],
        acc_sh.at[pl.ds(s * RPT + (RPT // CH) * CH, _tail)], gsem1).wait()
    start(1, 1)
    plsc.subcore_barrier()

    def body(i, carry):
        for b in range(_NBUF):
            j = _NBUF * i + b
            wait(b)
            scatter(j, b)

            @pl.when(j + _NBUF < NCHUNK)
            def _():
                start(j + _NBUF, b)
        return carry
    lax.fori_loop(0, NCHUNK // _NBUF, body, 0)

    for r in range(_NBUF * (NCHUNK // _NBUF), NCHUNK):
        wait(r % _NBUF)
        scatter(r, r % _NBUF)

    plsc.subcore_barrier()
    pltpu.sync_copy(acc_sh.at[pl.ds(s * RPT, RPT)],
                    parts_hbm.at[c, pl.ds(s * RPT, RPT)])


@jax.jit
def _agg_call(x, ei2d):
    f = functools.partial(
        pl.kernel,
        out_type=jax.ShapeDtypeStruct((NC, N, D), jnp.float32),
        mesh=_sc_mesh(),
        scratch_types=[
            pltpu.VMEM((NCHUNK, CH), jnp.int32),
            pltpu.VMEM((NCHUNK, CH), jnp.int32),
            pltpu.VMEM((CH, D), jnp.float32),
            pltpu.VMEM((CH, D), jnp.float32),
            pltpu.VMEM((CH, D), jnp.float32),
            pltpu.VMEM_SHARED((N, D), jnp.float32),
            pltpu.SemaphoreType.DMA,
            pltpu.SemaphoreType.DMA,
            pltpu.SemaphoreType.DMA,
        ],
        compiler_params=pltpu.CompilerParams(use_tc_tiling_on_sc=False),
    )(_agg_body)
    return f(x, ei2d)


def _y0_body(feat_ref, w0_ref, y0_ref):
    y0_ref[...] = jnp.dot(feat_ref[...], w0_ref[...],
                          preferred_element_type=jnp.float32)


@jax.jit
def _y0_call(feat, W0):
    return pl.pallas_call(
        _y0_body,
        grid=(N // BN,),
        in_specs=[
            pl.BlockSpec((BN, D), lambda i: (i, 0)),
            pl.BlockSpec((D, D), lambda i: (0, 0)),
        ],
        out_specs=pl.BlockSpec((BN, D), lambda i: (i, 0)),
        out_shape=jax.ShapeDtypeStruct((N, D), jnp.float32),
    )(feat, W0)


def _scale_body(y0_ref, dout_ref, x0_ref):
    ns = lax.rsqrt(jnp.maximum(dout_ref[0][:, :1], 1.0))
    x0_ref[...] = y0_ref[...] * ns


@jax.jit
def _scale_call(y0, deg2d):
    return pl.pallas_call(
        _scale_body,
        grid=(N // BN,),
        in_specs=[
            pl.BlockSpec((BN, D), lambda i: (i, 0)),
            pl.BlockSpec((1, BN, _LANES), lambda i: (0, i, 0)),
        ],
        out_specs=pl.BlockSpec((BN, D), lambda i: (i, 0)),
        out_shape=jax.ShapeDtypeStruct((N, D), jnp.float32),
    )(y0, deg2d)


def _mid_body(p0_ref, p1_ref, din_ref, dout_ref, b0_ref, alpha_ref, w1_ref,
              x1_ref, hg_ref):
    @pl.when(pl.program_id(0) == 0)
    def _():
        hg_ref[...] = jnp.zeros_like(hg_ref)

    nd = lax.rsqrt(jnp.maximum(din_ref[0][:, :1], 1.0))
    z = (p0_ref[0] + p1_ref[0]) * nd + b0_ref[...]
    a = alpha_ref[0, 0]
    h = jnp.where(z >= 0, z, a * z)
    hg_ref[...] += jnp.sum(h, axis=0, keepdims=True)
    ns = lax.rsqrt(jnp.maximum(dout_ref[0][:, :1], 1.0))
    x1_ref[...] = jnp.dot(h, w1_ref[...],
                          preferred_element_type=jnp.float32) * ns


@jax.jit
def _mid_call(p, deg2d, b0, alpha, W1):
    return pl.pallas_call(
        _mid_body,
        grid=(N // BN,),
        in_specs=[
            pl.BlockSpec((1, BN, D), lambda i: (0, i, 0)),
            pl.BlockSpec((1, BN, D), lambda i: (1, i, 0)),
            pl.BlockSpec((1, BN, _LANES), lambda i: (1, i, 0)),
            pl.BlockSpec((1, BN, _LANES), lambda i: (0, i, 0)),
            pl.BlockSpec((1, D), lambda i: (0, 0)),
            pl.BlockSpec((1, 1), lambda i: (0, 0)),
            pl.BlockSpec((D, D), lambda i: (0, 0)),
        ],
        out_specs=[
            pl.BlockSpec((BN, D), lambda i: (i, 0)),
            pl.BlockSpec((1, D), lambda i: (0, 0)),
        ],
        out_shape=[
            jax.ShapeDtypeStruct((N, D), jnp.float32),
            jax.ShapeDtypeStruct((1, D), jnp.float32),
        ],
    )(p, p, deg2d, deg2d, b0, alpha, W1)


def _out_body(q0_ref, q1_ref, din_ref, b1_ref, alpha_ref, h2_ref, hg_ref):
    @pl.when(pl.program_id(0) == 0)
    def _():
        hg_ref[...] = jnp.zeros_like(hg_ref)

    nd = lax.rsqrt(jnp.maximum(din_ref[0][:, :1], 1.0))
    z = (q0_ref[0] + q1_ref[0]) * nd + b1_ref[...]
    a = alpha_ref[0, 0]
    h = jnp.where(z >= 0, z, a * z)
    h2_ref[...] = h
    hg_ref[...] += jnp.sum(h, axis=0, keepdims=True)


@jax.jit
def _out_call(q, deg2d, b1, alpha):
    return pl.pallas_call(
        _out_body,
        grid=(N // BN,),
        in_specs=[
            pl.BlockSpec((1, BN, D), lambda i: (0, i, 0)),
            pl.BlockSpec((1, BN, D), lambda i: (1, i, 0)),
            pl.BlockSpec((1, BN, _LANES), lambda i: (1, i, 0)),
            pl.BlockSpec((1, D), lambda i: (0, 0)),
            pl.BlockSpec((1, 1), lambda i: (0, 0)),
        ],
        out_specs=[
            pl.BlockSpec((BN, D), lambda i: (i, 0)),
            pl.BlockSpec((1, D), lambda i: (0, 0)),
        ],
        out_shape=[
            jax.ShapeDtypeStruct((N, D), jnp.float32),
            jax.ShapeDtypeStruct((1, D), jnp.float32),
        ],
    )(q, q, deg2d, b1, alpha)


def kernel(feat, edge_index, W0, b0, W1, b1, alpha):
    ei2d = edge_index.reshape(2 * (E // CH), CH)
    parts = _degree_call(ei2d)
    deg2d = _norm_call(parts)
    y0 = _y0_call(feat, W0)
    x0 = _scale_call(y0, deg2d)
    p = _agg_call(x0, ei2d)
    x1, hg0 = _mid_call(p, deg2d, b0.reshape(1, D), alpha.reshape(1, 1), W1)
    q = _agg_call(x1, ei2d)
    h2, hg1 = _out_call(q, deg2d, b1.reshape(1, D), alpha.reshape(1, 1))
    hg = jnp.concatenate((hg0, hg1), axis=-1)
    return (h2, hg)


# speedup vs baseline: 14.1259x; 1.0283x over previous
import functools

import jax
import jax.numpy as jnp
from jax import lax
from jax.experimental import pallas as pl
from jax.experimental.pallas import tpu as pltpu
from jax.experimental.pallas import tpu_sc as plsc

N = 10000
E = 320000
D = 128

NC = 2
NS = 16
NW = NC * NS
N_PAD = 10240
EW = E // NW
CH = 80
NCHUNK = EW // CH
SL = N_PAD // NW
RPT = N // NS
BN = 2000

_LANES = 16


def _sc_mesh():
    return plsc.VectorSubcoreMesh(core_axis_name="c", subcore_axis_name="s")


def _degree_body(ei_hbm, parts_hbm, sidx2, didx2, hout_v, hin_v, sem):
    c = lax.axis_index("c")
    s = lax.axis_index("s")
    wid = c * NS + s
    ones = jnp.ones((_LANES,), jnp.float32)
    zeros = jnp.zeros((_LANES,), jnp.float32)

    pltpu.async_copy(ei_hbm.at[pl.ds(wid * NCHUNK, NCHUNK)], sidx2, sem)
    pltpu.async_copy(ei_hbm.at[pl.ds(E // CH + wid * NCHUNK, NCHUNK)],
                     didx2, sem)

    def zero_body(i, carry):
        hout_v[pl.ds(i * _LANES, _LANES)] = zeros
        hin_v[pl.ds(i * _LANES, _LANES)] = zeros
        return carry
    lax.fori_loop(0, N_PAD // _LANES, zero_body, 0)

    pltpu.make_async_copy(ei_hbm.at[pl.ds(wid * NCHUNK, NCHUNK)],
                          sidx2, sem).wait()
    pltpu.make_async_copy(ei_hbm.at[pl.ds(E // CH + wid * NCHUNK, NCHUNK)],
                          didx2, sem).wait()

    def acc_body(t, carry):
        r = t // (CH // _LANES)
        k = t % (CH // _LANES)
        plsc.addupdate_scatter(hout_v, [sidx2[r, pl.ds(k * _LANES, _LANES)]],
                               ones)
        plsc.addupdate_scatter(hin_v, [didx2[r, pl.ds(k * _LANES, _LANES)]],
                               ones)
        return carry
    lax.fori_loop(0, EW // _LANES, acc_body, 0)
    pltpu.sync_copy(hout_v, parts_hbm.at[wid])
    pltpu.sync_copy(hin_v, parts_hbm.at[NW + wid])


@jax.jit
def _degree_call(ei2d):
    f = functools.partial(
        pl.kernel,
        out_type=jax.ShapeDtypeStruct((2 * NW, N_PAD), jnp.float32),
        mesh=_sc_mesh(),
        scratch_types=[
            pltpu.VMEM((NCHUNK, CH), jnp.int32),
            pltpu.VMEM((NCHUNK, CH), jnp.int32),
            pltpu.VMEM((N_PAD,), jnp.float32),
            pltpu.VMEM((N_PAD,), jnp.float32),
            pltpu.SemaphoreType.DMA,
        ],
        compiler_params=pltpu.CompilerParams(
            needs_layout_passes=False, use_tc_tiling_on_sc=False),
    )(_degree_body)
    return f(ei2d)


def _norm_body(parts_hbm, deg2d_hbm, pbuf, acc, rowbuf, sem):
    c = lax.axis_index("c")
    s = lax.axis_index("s")
    wid = c * NS + s
    n0 = wid * SL
    zeros = jnp.zeros((_LANES,), jnp.float32)

    def load_body(t, carry):
        pltpu.async_copy(parts_hbm.at[t, pl.ds(n0, SL)], pbuf.at[t], sem)
        return carry
    lax.fori_loop(0, 2 * NW, load_body, 0)

    def wait_body(t, carry):
        pltpu.make_async_copy(parts_hbm.at[t, pl.ds(n0, SL)], pbuf.at[t], sem).wait()
        return carry
    lax.fori_loop(0, 2 * NW, wait_body, 0)

    for a in (0, 1):
        def zero_body(k, carry):
            acc[pl.ds(k * _LANES, _LANES)] = zeros
            return carry
        lax.fori_loop(0, SL // _LANES, zero_body, 0)

        def red_body(i, carry):
            t = i // (SL // _LANES)
            k = i % (SL // _LANES)
            acc[pl.ds(k * _LANES, _LANES)] = (
                acc[pl.ds(k * _LANES, _LANES)]
                + pbuf[a * NW + t, pl.ds(k * _LANES, _LANES)]
            )
            return carry
        lax.fori_loop(0, NW * (SL // _LANES), red_body, 0)

        def bcast_body(g, carry):
            vec16 = acc[pl.ds(g * _LANES, _LANES)]
            for j in range(_LANES):
                rowbuf[g * _LANES + j, pl.ds(0, _LANES)] = lax.broadcast(
                    vec16[j], (_LANES,))
            return carry
        lax.fori_loop(0, SL // _LANES, bcast_body, 0)
        pltpu.sync_copy(rowbuf,
                        deg2d_hbm.at[a, pl.ds(n0, SL), pl.ds(0, _LANES)])


@jax.jit
def _norm_call(parts):
    f = functools.partial(
        pl.kernel,
        out_type=jax.ShapeDtypeStruct((2, N_PAD, D), jnp.float32),
        mesh=_sc_mesh(),
        scratch_types=[
            pltpu.VMEM((2 * NW, SL), jnp.float32),
            pltpu.VMEM((SL,), jnp.float32),
            pltpu.VMEM((SL, _LANES), jnp.float32),
            pltpu.SemaphoreType.DMA,
        ],
        compiler_params=pltpu.CompilerParams(use_tc_tiling_on_sc=False),
    )(_norm_body)
    return f(parts)


_NBUF = 3


def _agg_body(x_hbm, ei_hbm, parts_hbm,
              sidx2d, didx2d, rows0, rows1, rows2, acc_sh,
              gsem0, gsem1, gsem2):
    c = lax.axis_index("c")
    s = lax.axis_index("s")
    wid = c * NS + s
    rows = (rows0, rows1, rows2)
    gsem = (gsem0, gsem1, gsem2)

    pltpu.async_copy(ei_hbm.at[pl.ds(wid * NCHUNK, NCHUNK)], sidx2d, gsem2)
    pltpu.async_copy(ei_hbm.at[pl.ds(E // CH + wid * NCHUNK, NCHUNK)],
                     didx2d, gsem2)

    zv = jnp.zeros((_LANES,), jnp.float32)

    def zero_body(i, carry):
        rows1[i // (D // _LANES),
              pl.ds((i % (D // _LANES)) * _LANES, _LANES)] = zv
        return carry
    lax.fori_loop(0, CH * (D // _LANES), zero_body, 0)
    _tail = RPT - (RPT // CH) * CH
    for k in range(RPT // CH):
        pltpu.async_copy(rows1, acc_sh.at[pl.ds(s * RPT + k * CH, CH)], gsem1)
    pltpu.async_copy(rows1.at[pl.ds(0, _tail)],
                     acc_sh.at[pl.ds(s * RPT + (RPT // CH) * CH, _tail)],
                     gsem1)

    def start(j, b):
        pltpu.async_copy(x_hbm.at[sidx2d.at[j]], rows[b], gsem[b])

    def wait(b):
        pltpu.make_async_copy(x_hbm.at[sidx2d.at[0]], rows[b], gsem[b]).wait()

    def scatter(j, b):
        pltpu.sync_copy(rows[b], acc_sh.at[didx2d.at[j]], add=True)

    pltpu.make_async_copy(
        ei_hbm.at[pl.ds(wid * NCHUNK, NCHUNK)], sidx2d, gsem2).wait()
    pltpu.make_async_copy(
        ei_hbm.at[pl.ds(E // CH + wid * NCHUNK, NCHUNK)], didx2d, gsem2).wait()
    start(0, 0)
    start(2, 2)
    for k in range(RPT // CH):
        pltpu.make_async_copy(
            rows1, acc_sh.at[pl.ds(s * RPT + k * CH, CH)], gsem1).wait()
    pltpu.make_async_copy(
        rows1.at[pl.ds(0, _tail)],
        acc_sh.at[pl.ds(s * RPT + (RPT // CH) * CH, _tail)], gsem1).wait()
    start(1, 1)
    plsc.subcore_barrier()

    def body(i, carry):
        for b in range(_NBUF):
            j = _NBUF * i + b
            wait(b)
            scatter(j, b)

            @pl.when(j + _NBUF < NCHUNK)
            def _():
                start(j + _NBUF, b)
        return carry
    lax.fori_loop(0, NCHUNK // _NBUF, body, 0)

    for r in range(_NBUF * (NCHUNK // _NBUF), NCHUNK):
        wait(r % _NBUF)
        scatter(r, r % _NBUF)

    plsc.subcore_barrier()
    pltpu.sync_copy(acc_sh.at[pl.ds(s * RPT, RPT)],
                    parts_hbm.at[c, pl.ds(s * RPT, RPT)])


@jax.jit
def _agg_call(x, ei2d):
    f = functools.partial(
        pl.kernel,
        out_type=jax.ShapeDtypeStruct((NC, N, D), jnp.float32),
        mesh=_sc_mesh(),
        scratch_types=[
            pltpu.VMEM((NCHUNK, CH), jnp.int32),
            pltpu.VMEM((NCHUNK, CH), jnp.int32),
            pltpu.VMEM((CH, D), jnp.float32),
            pltpu.VMEM((CH, D), jnp.float32),
            pltpu.VMEM((CH, D), jnp.float32),
            pltpu.VMEM_SHARED((N, D), jnp.float32),
            pltpu.SemaphoreType.DMA,
            pltpu.SemaphoreType.DMA,
            pltpu.SemaphoreType.DMA,
        ],
        compiler_params=pltpu.CompilerParams(use_tc_tiling_on_sc=False),
    )(_agg_body)
    return f(x, ei2d)


def _y0_body(feat_ref, w0_ref, y0_ref):
    y0_ref[...] = jnp.dot(feat_ref[...], w0_ref[...],
                          preferred_element_type=jnp.float32)


@jax.jit
def _y0_call(feat, W0):
    return pl.pallas_call(
        _y0_body,
        grid=(N // BN,),
        in_specs=[
            pl.BlockSpec((BN, D), lambda i: (i, 0)),
            pl.BlockSpec((D, D), lambda i: (0, 0)),
        ],
        out_specs=pl.BlockSpec((BN, D), lambda i: (i, 0)),
        out_shape=jax.ShapeDtypeStruct((N, D), jnp.float32),
    )(feat, W0)


def _scale_body(y0_ref, dout_ref, x0_ref):
    ns = lax.rsqrt(jnp.maximum(dout_ref[0][:, :1], 1.0))
    x0_ref[...] = y0_ref[...] * ns


@jax.jit
def _scale_call(y0, deg2d):
    return pl.pallas_call(
        _scale_body,
        grid=(N // BN,),
        in_specs=[
            pl.BlockSpec((BN, D), lambda i: (i, 0)),
            pl.BlockSpec((1, BN, D), lambda i: (0, i, 0)),
        ],
        out_specs=pl.BlockSpec((BN, D), lambda i: (i, 0)),
        out_shape=jax.ShapeDtypeStruct((N, D), jnp.float32),
    )(y0, deg2d)


def _mid_body(p0_ref, p1_ref, din_ref, dout_ref, b0_ref, alpha_ref, w1_ref,
              x1_ref, hg_ref):
    @pl.when(pl.program_id(0) == 0)
    def _():
        hg_ref[...] = jnp.zeros_like(hg_ref)

    nd = lax.rsqrt(jnp.maximum(din_ref[0][:, :1], 1.0))
    z = (p0_ref[0] + p1_ref[0]) * nd + b0_ref[...]
    a = alpha_ref[0, 0]
    h = jnp.where(z >= 0, z, a * z)
    hg_ref[...] += jnp.sum(h, axis=0, keepdims=True)
    ns = lax.rsqrt(jnp.maximum(dout_ref[0][:, :1], 1.0))
    x1_ref[...] = jnp.dot(h, w1_ref[...],
                          preferred_element_type=jnp.float32) * ns


@jax.jit
def _mid_call(p, deg2d, b0, alpha, W1):
    return pl.pallas_call(
        _mid_body,
        grid=(N // BN,),
        in_specs=[
            pl.BlockSpec((1, BN, D), lambda i: (0, i, 0)),
            pl.BlockSpec((1, BN, D), lambda i: (1, i, 0)),
            pl.BlockSpec((1, BN, D), lambda i: (1, i, 0)),
            pl.BlockSpec((1, BN, D), lambda i: (0, i, 0)),
            pl.BlockSpec((1, D), lambda i: (0, 0)),
            pl.BlockSpec((1, 1), lambda i: (0, 0)),
            pl.BlockSpec((D, D), lambda i: (0, 0)),
        ],
        out_specs=[
            pl.BlockSpec((BN, D), lambda i: (i, 0)),
            pl.BlockSpec((1, D), lambda i: (0, 0)),
        ],
        out_shape=[
            jax.ShapeDtypeStruct((N, D), jnp.float32),
            jax.ShapeDtypeStruct((1, D), jnp.float32),
        ],
    )(p, p, deg2d, deg2d, b0, alpha, W1)


def _out_body(q0_ref, q1_ref, din_ref, b1_ref, alpha_ref, h2_ref, hg_ref):
    @pl.when(pl.program_id(0) == 0)
    def _():
        hg_ref[...] = jnp.zeros_like(hg_ref)

    nd = lax.rsqrt(jnp.maximum(din_ref[0][:, :1], 1.0))
    z = (q0_ref[0] + q1_ref[0]) * nd + b1_ref[...]
    a = alpha_ref[0, 0]
    h = jnp.where(z >= 0, z, a * z)
    h2_ref[...] = h
    hg_ref[...] += jnp.sum(h, axis=0, keepdims=True)


@jax.jit
def _out_call(q, deg2d, b1, alpha):
    return pl.pallas_call(
        _out_body,
        grid=(N // BN,),
        in_specs=[
            pl.BlockSpec((1, BN, D), lambda i: (0, i, 0)),
            pl.BlockSpec((1, BN, D), lambda i: (1, i, 0)),
            pl.BlockSpec((1, BN, D), lambda i: (1, i, 0)),
            pl.BlockSpec((1, D), lambda i: (0, 0)),
            pl.BlockSpec((1, 1), lambda i: (0, 0)),
        ],
        out_specs=[
            pl.BlockSpec((BN, D), lambda i: (i, 0)),
            pl.BlockSpec((1, D), lambda i: (0, 0)),
        ],
        out_shape=[
            jax.ShapeDtypeStruct((N, D), jnp.float32),
            jax.ShapeDtypeStruct((1, D), jnp.float32),
        ],
    )(q, q, deg2d, b1, alpha)


def kernel(feat, edge_index, W0, b0, W1, b1, alpha):
    ei2d = edge_index.reshape(2 * (E // CH), CH)
    parts = _degree_call(ei2d)
    deg2d = _norm_call(parts)
    y0 = _y0_call(feat, W0)
    x0 = _scale_call(y0, deg2d)
    p = _agg_call(x0, ei2d)
    x1, hg0 = _mid_call(p, deg2d, b0.reshape(1, D), alpha.reshape(1, 1), W1)
    q = _agg_call(x1, ei2d)
    h2, hg1 = _out_call(q, deg2d, b1.reshape(1, D), alpha.reshape(1, 1))
    hg = jnp.concatenate((hg0, hg1), axis=-1)
    return (h2, hg)
